# Initial kernel scaffold; baseline (speedup 1.0000x reference)
import jax
import jax.numpy as jnp
from jax.experimental import pallas as pl


def kernel(input_ids, scores):
    raise NotImplementedError("write your pallas kernel here")


# speedup vs baseline: 1.1332x; 1.1332x over previous
import numpy as np
import jax
import jax.numpy as jnp
from jax.experimental import pallas as pl
from jax.experimental.pallas import tpu as pltpu

_SEED = 42
_SAMPLE_SEED = 123
_NGRAM = 4
_BIAS = 2.0
_TOP_P = 0.9
_V = 1000000
_LANES = 15625
_SUBS = 64
_SUB = 8
_NCH = _SUBS // _SUB
_CHUNK = _SUB * _LANES
_TINY = float(np.finfo(np.float32).tiny)


def _i32(x):
    return jnp.int32(int(np.uint32(x).astype(np.int32)))


def _lrs(x, s):
    return (x >> s) & ((1 << (32 - s)) - 1)


def _rotl(x, r):
    return (x << r) | _lrs(x, 32 - r)


def _threefry_bits(k0, k1, cnt):
    ks2 = k0 ^ k1 ^ _i32(0x1BD11BDA)
    ks = (k0, k1, ks2)
    rot = ((13, 15, 26, 6), (17, 29, 16, 24))
    x0 = jnp.zeros_like(cnt) + k0
    x1 = cnt + k1
    for i in range(5):
        for r in rot[i % 2]:
            x0 = x0 + x1
            x1 = _rotl(x1, r) ^ x0
        x0 = x0 + ks[(i + 1) % 3]
        x1 = x1 + ks[(i + 2) % 3] + _i32(i + 1)
    return x0 ^ x1


def _chunk_iotas(c):
    s_io = jax.lax.broadcasted_iota(jnp.int32, (1, _SUB, _LANES), 1)
    l_io = jax.lax.broadcasted_iota(jnp.int32, (1, _SUB, _LANES), 2)
    return (c * _SUB + s_io) * _LANES + l_io


def _green_kernel(gk_ref, scores_ref, out_ref):
    r = pl.program_id(0)
    c = pl.program_id(1)
    cnt = _chunk_iotas(c)
    bits = _threefry_bits(gk_ref[r, 0], gk_ref[r, 1], cnt)
    green = (bits & 1).astype(jnp.float32)
    out_ref[...] = scores_ref[...] + jnp.float32(_BIAS) * green


def _sample_kernel(sk_ref, sp_ref, ord_ref, tok_ref, fstate, istate):
    r = pl.program_id(0)
    c = pl.program_id(1)

    @pl.when(c == 0)
    def _():
        fstate[0] = 0.0
        fstate[1] = -1e30
        istate[0] = _i32(_V)
        istate[1] = 0

    p = sp_ref[...]
    gidx = _chunk_iotas(c)
    jloc = gidx - c * _CHUNK

    s_prev = fstate[0]
    t_chunk = jnp.sum(p)
    topp = jnp.float32(_TOP_P)

    @pl.when((istate[0] == _i32(_V)) & (s_prev + t_chunk >= topp))
    def _():
        def body(_, lohi):
            lo, hi = lohi
            mid = (lo + hi) >> 1
            sm = s_prev + jnp.sum(jnp.where(jloc <= mid, p, 0.0))
            pred = sm >= topp
            return jnp.where(pred, lo, mid + 1), jnp.where(pred, mid, hi)

        lo, _hi = jax.lax.fori_loop(
            0, 17, body, (jnp.int32(0), jnp.int32(_CHUNK - 1)))
        istate[0] = c * _CHUNK + lo

    fstate[0] = s_prev + t_chunk

    cutoff = istate[0]
    kept = gidx <= cutoff
    bits = _threefry_bits(sk_ref[r, 0], sk_ref[r, 1], gidx)
    fb = _lrs(bits, 9) | _i32(0x3F800000)
    u = jax.lax.bitcast_convert_type(fb, jnp.float32) - jnp.float32(1.0)
    g = -jnp.log(-jnp.log(jnp.maximum(jnp.float32(_TINY), u)))
    lp = jnp.log(jnp.where(kept, p, jnp.float32(1.0)))
    score = jnp.where(kept, lp + g, jnp.float32(-jnp.inf))
    m = jnp.max(score)

    @pl.when(m > fstate[1])
    def _():
        pos = jnp.min(jnp.where(score == m, gidx, _i32(0x7FFFFFFF)))
        tok = jnp.sum(jnp.where(gidx == pos, ord_ref[...], 0))
        fstate[1] = m
        istate[1] = tok

    @pl.when(c == _NCH - 1)
    def _():
        tok_ref[...] = jnp.full((1, 8, 128), istate[1], jnp.int32)


def _scatter_kernel(tok_ref, out_ref):
    r = pl.program_id(0)
    c = pl.program_id(1)
    t = _chunk_iotas(c)
    out_ref[...] = jnp.where(t == tok_ref[r], jnp.float32(100000.0),
                             jnp.float32(1e-05))


def _block_spec():
    return pl.BlockSpec((1, _SUB, _LANES), lambda r, c: (r, c, 0))


def kernel(input_ids, scores):
    B, V = scores.shape
    ids = input_ids[:, -(_NGRAM - 1):]
    row_sums = jnp.sum(ids, axis=1).astype(jnp.int32)
    base_key = jax.random.key(_SEED)
    gkeys = jax.vmap(lambda rs: jax.random.fold_in(base_key, rs))(row_sums)
    gk2 = jax.vmap(lambda k: jax.random.split(k)[1])(gkeys)
    gkd = jax.lax.bitcast_convert_type(jax.random.key_data(gk2), jnp.int32)
    skeys = jax.random.split(jax.random.key(_SAMPLE_SEED), B)
    skd = jax.lax.bitcast_convert_type(jax.random.key_data(skeys), jnp.int32)

    scores3 = scores.reshape(B, _SUBS, _LANES)
    smem = pl.BlockSpec(memory_space=pltpu.SMEM)

    biased = pl.pallas_call(
        _green_kernel,
        grid=(B, _NCH),
        in_specs=[smem, _block_spec()],
        out_specs=_block_spec(),
        out_shape=jax.ShapeDtypeStruct((B, _SUBS, _LANES), jnp.float32),
        compiler_params=pltpu.CompilerParams(
            dimension_semantics=("parallel", "parallel")),
    )(gkd, scores3)

    probs = jax.nn.softmax(biased.reshape(B, V), axis=-1)
    iota = jnp.broadcast_to(jnp.arange(V, dtype=jnp.int32), (B, V))
    sneg, order = jax.lax.sort((-probs, iota), dimension=1,
                               is_stable=True, num_keys=1)
    sorted_p = (-sneg).reshape(B, _SUBS, _LANES)
    order3 = order.reshape(B, _SUBS, _LANES)

    tok_tile = pl.pallas_call(
        _sample_kernel,
        grid=(B, _NCH),
        in_specs=[smem, _block_spec(), _block_spec()],
        out_specs=pl.BlockSpec((1, 8, 128), lambda r, c: (r, 0, 0)),
        out_shape=jax.ShapeDtypeStruct((B, 8, 128), jnp.int32),
        scratch_shapes=[pltpu.SMEM((2,), jnp.float32),
                        pltpu.SMEM((2,), jnp.int32)],
        compiler_params=pltpu.CompilerParams(
            dimension_semantics=("parallel", "arbitrary")),
    )(skd, sorted_p, order3)
    tokens = tok_tile[:, 0, 0]

    out = pl.pallas_call(
        _scatter_kernel,
        grid=(B, _NCH),
        in_specs=[smem],
        out_specs=_block_spec(),
        out_shape=jax.ShapeDtypeStruct((B, _SUBS, _LANES), jnp.float32),
        compiler_params=pltpu.CompilerParams(
            dimension_semantics=("parallel", "parallel")),
    )(tokens)
    return out.reshape(B, V)
